# baseline (speedup 1.0000x reference)
import functools

import jax
import jax.numpy as jnp
from jax import lax
from jax.experimental import pallas as pl
from jax.experimental.pallas import tpu as pltpu
from jax.experimental.pallas import tpu_sc as plsc

N = 10000
D = 128
E = 320000
NC, NS = 2, 16
CHUNK = 40
CHUNK1 = 40
NP_ = 10240
RPT = NP_ // NS
EPS = 1e-5
BR = 1000


def _fill_ones(ref, chunk):
    v = jnp.ones((16,), jnp.float32)
    for i in range(chunk):
        for j in range(D // 16):
            ref[i, pl.ds(j * 16, 16)] = v


NBUF = 5


def _feature_pass(table, eint, dst, acc, bufs, coff, chunk, n):
    idxc, rows, sic, sg, ssc = bufs
    G = n // NBUF

    def s_idx(b):
        return idxc[b].at[pl.ds(0, chunk)]

    def d_idx(b):
        return idxc[b].at[pl.ds(chunk, chunk)]

    def issue_idx(i, b):
        pltpu.async_copy(eint.at[pl.ds(coff(i), 2 * chunk)], idxc[b], sic[b])

    def wait_sc(b):
        pltpu.make_async_copy(rows[b], acc.at[d_idx(b)], ssc[b]).wait()

    def wait_g(b):
        pltpu.make_async_copy(table.at[s_idx(b)], rows[b], sg[b]).wait()

    def launch_g(i, b):
        pltpu.make_async_copy(eint.at[pl.ds(0, 2 * chunk)], idxc[b],
                              sic[b]).wait()
        pltpu.async_copy(table.at[s_idx(b)], rows[b], sg[b])

    issue_idx(0, 0)
    issue_idx(1, 1)
    launch_g(0, 0)

    def group(g, carry):
        for b in range(NBUF):
            b1, b2 = (b + 1) % NBUF, (b + 2) % NBUF
            i = NBUF * g + b
            if b >= NBUF - 2:
                wait_sc(b2)
            else:
                @pl.when(g > 0)
                def _():
                    wait_sc(b2)
            if b < NBUF - 2:
                issue_idx(i + 2, b2)
            else:
                @pl.when(g < G - 1)
                def _():
                    issue_idx(i + 2, b2)
            if b < NBUF - 1:
                launch_g(i + 1, b1)
            else:
                @pl.when(g < G - 1)
                def _():
                    launch_g(i + 1, b1)
            wait_g(b)
            pltpu.async_copy(rows[b], acc.at[d_idx(b)], ssc[b], add=True)
        return carry

    lax.fori_loop(0, G, group, 0)
    for j in range(NBUF - 2):
        wait_sc((2 + j) % NBUF)


def _count_pass(eint, acc, ones, idxc, sic, ssc, coff, chunk, n):
    G = n // NBUF

    def d_idx(b):
        return idxc[b].at[pl.ds(chunk, chunk)]

    def wait_sc(b):
        pltpu.make_async_copy(ones, acc.at[d_idx(b)], ssc[b]).wait()

    def issue_idx(i, b):
        pltpu.async_copy(eint.at[pl.ds(coff(i), 2 * chunk)], idxc[b], sic[b])

    issue_idx(0, 0)

    def group(g, carry):
        for b in range(NBUF):
            b1 = (b + 1) % NBUF
            i = NBUF * g + b
            if b == NBUF - 1:
                wait_sc(b1)
            else:
                @pl.when(g > 0)
                def _():
                    wait_sc(b1)
            if b < NBUF - 1:
                issue_idx(i + 1, b1)
            else:
                @pl.when(g < G - 1)
                def _():
                    issue_idx(i + 1, b1)
            pltpu.make_async_copy(eint.at[pl.ds(0, 2 * chunk)], idxc[b],
                                  sic[b]).wait()
            pltpu.async_copy(ones, acc.at[d_idx(b)], ssc[b], add=True)
        return carry

    lax.fori_loop(0, G, group, 0)
    for j in range(NBUF - 1):
        wait_sc((1 + j) % NBUF)


def _sc_scratch(chunk):
    vm = []
    for _ in range(NBUF):
        vm.append(pltpu.VMEM((2 * chunk,), jnp.int32))
    for _ in range(NBUF):
        vm.append(pltpu.VMEM((chunk, D), jnp.float32))
    for _ in range(3 * NBUF):
        vm.append(pltpu.SemaphoreType.DMA)
    vm.append(pltpu.VMEM_SHARED((NP_, D), jnp.float32))
    return vm


def _sc_layer0(x_bert, x_doc, eint_bd, eint_db, zf):
    mesh = plsc.VectorSubcoreMesh(core_axis_name="c", subcore_axis_name="s")
    ept = E // NS
    n = ept // CHUNK

    @functools.partial(
        pl.kernel,
        out_type=[
            jax.ShapeDtypeStruct((NP_, D), jnp.float32),
            jax.ShapeDtypeStruct((NP_, D), jnp.float32),
            jax.ShapeDtypeStruct((NP_, D), jnp.float32),
            jax.ShapeDtypeStruct((NP_, D), jnp.float32),
        ],
        mesh=mesh,
        scratch_types=_sc_scratch(CHUNK),
    )
    def k(xb, xd, ebd, edb, zfeat, o_sd, o_cd, o_sb, o_cb, *scr):
        idxc = scr[0:NBUF]
        rows = scr[NBUF:2 * NBUF]
        sic = scr[2 * NBUF:3 * NBUF]
        sg = scr[3 * NBUF:4 * NBUF]
        ssc = scr[4 * NBUF:5 * NBUF]
        acc = scr[5 * NBUF]
        bufs = (idxc, rows, sic, sg, ssc)
        cid = lax.axis_index("c")
        sid_ = lax.axis_index("s")
        r0 = sid_ * RPT
        pltpu.sync_copy(zfeat.at[pl.ds(r0, RPT)], acc.at[pl.ds(r0, RPT)])
        plsc.subcore_barrier()

        def export(dst_hbm):
            pltpu.sync_copy(acc.at[pl.ds(r0, RPT)],
                            dst_hbm.at[pl.ds(r0, RPT)])

        def coff(i):
            return (sid_ * n + i) * 2 * CHUNK

        for c_sel, (table, eint, o_s, o_c) in enumerate(
                ((xb, ebd, o_sd, o_cd), (xd, edb, o_sb, o_cb))):
            @pl.when(cid == c_sel)
            def _():
                _feature_pass(table, eint, None, acc, bufs, coff, CHUNK, n)
                plsc.subcore_barrier()
                export(o_s)
                plsc.subcore_barrier()
                pltpu.sync_copy(zfeat.at[pl.ds(r0, RPT)],
                                acc.at[pl.ds(r0, RPT)])
                _fill_ones(rows[0], CHUNK)
                plsc.subcore_barrier()
                _count_pass(eint, acc, rows[0], idxc, sic, ssc,
                            coff, CHUNK, n)
                plsc.subcore_barrier()
                export(o_c)

    return k(x_bert, x_doc, eint_bd, eint_db, zf)


def _sc_layer1(x_doc1, eint_db, zf):
    mesh = plsc.VectorSubcoreMesh(core_axis_name="c", subcore_axis_name="s")
    ept = E // (NC * NS)
    n = ept // CHUNK1

    @functools.partial(
        pl.kernel,
        out_type=[jax.ShapeDtypeStruct((NP_, D), jnp.float32),
                  jax.ShapeDtypeStruct((NP_, D), jnp.float32)],
        mesh=mesh,
        scratch_types=_sc_scratch(CHUNK1),
    )
    def k(xd1, edb, zfeat, o_s0, o_s1, *scr):
        idxc = scr[0:NBUF]
        rows = scr[NBUF:2 * NBUF]
        sic = scr[2 * NBUF:3 * NBUF]
        sg = scr[3 * NBUF:4 * NBUF]
        ssc = scr[4 * NBUF:5 * NBUF]
        acc = scr[5 * NBUF]
        bufs = (idxc, rows, sic, sg, ssc)
        cid = lax.axis_index("c")
        sid_ = lax.axis_index("s")
        r0 = sid_ * RPT
        pltpu.sync_copy(zfeat.at[pl.ds(r0, RPT)], acc.at[pl.ds(r0, RPT)])
        plsc.subcore_barrier()

        wid = cid * NS + sid_

        def coff(i):
            return (wid * n + i) * 2 * CHUNK1

        _feature_pass(xd1, edb, None, acc, bufs, coff, CHUNK1, n)
        plsc.subcore_barrier()

        for c_sel, o_s in enumerate((o_s0, o_s1)):
            @pl.when(cid == c_sel)
            def _():
                pltpu.sync_copy(acc.at[pl.ds(r0, RPT)],
                                o_s.at[pl.ds(r0, RPT)])

    return k(x_doc1, eint_db, zf)


def _leaky(x):
    return jnp.where(x >= 0, x, 0.1 * x)


def _tc_layer0(S_d, C_d, S_b, C_b, x_bert, x_doc,
               W0_bd_l, b0_bd_l, W0_bd_r, W0_db_l, b0_db_l, W0_db_r,
               bn0_g, bn0_b):
    def body(sd, cd, sb, cb, xb, xd, wbl, bbl, wbr, wdl, bdl, wdr, g, b,
             o_xd1, o_xb1):
        scale = g[...] / jnp.sqrt(jnp.float32(1.0 + EPS))
        shift = b[...]
        cdv = jnp.maximum(cd[...][:, 0:1], 1.0)
        nd = (jnp.dot(sd[...], wbl[...], preferred_element_type=jnp.float32)
              / cdv + bbl[...]
              + jnp.dot(xd[...], wbr[...], preferred_element_type=jnp.float32))
        o_xd1[...] = _leaky(nd) * scale + shift
        cbv = jnp.maximum(cb[...][:, 0:1], 1.0)
        nb = (jnp.dot(sb[...], wdl[...], preferred_element_type=jnp.float32)
              / cbv + bdl[...]
              + jnp.dot(xb[...], wdr[...], preferred_element_type=jnp.float32))
        o_xb1[...] = _leaky(nb) * scale + shift

    row = pl.BlockSpec((BR, D), lambda i: (i, 0))
    wspec = pl.BlockSpec((D, D), lambda i: (0, 0))
    vspec = pl.BlockSpec((1, D), lambda i: (0, 0))
    return pl.pallas_call(
        body,
        grid=(N // BR,),
        in_specs=[row, row, row, row, row, row,
                  wspec, vspec, wspec, wspec, vspec, wspec, vspec, vspec],
        out_specs=[row, row],
        out_shape=[jax.ShapeDtypeStruct((N, D), jnp.float32),
                   jax.ShapeDtypeStruct((N, D), jnp.float32)],
    )(S_d, C_d, S_b, C_b, x_bert, x_doc,
      W0_bd_l, b0_bd_l.reshape(1, D), W0_bd_r,
      W0_db_l, b0_db_l.reshape(1, D), W0_db_r,
      bn0_g.reshape(1, D), bn0_b.reshape(1, D))


def _tc_head(S1a, S1b, C_b, x_bert1,
             W1_db_l, b1_db_l, W1_db_r, bn1_g, bn1_b,
             lin1_W, lin1_b, lin2_W, lin2_b):
    H2 = D // 2
    OUT = 16

    def body(s0, s1, cb, xb1, wdl, bdl, wdr, g, b, l1w, l1b, l2w, l2b, o):
        scale = g[...] / jnp.sqrt(jnp.float32(1.0 + EPS))
        cbv = jnp.maximum(cb[...][:, 0:1], 1.0)
        s = s0[...] + s1[...]
        nb = (jnp.dot(s, wdl[...], preferred_element_type=jnp.float32)
              / cbv + bdl[...]
              + jnp.dot(xb1[...], wdr[...], preferred_element_type=jnp.float32))
        xb2 = _leaky(nb) * scale + b[...]
        h = _leaky(jnp.dot(xb2, l1w[...], preferred_element_type=jnp.float32)
                   + l1b[...])
        o[...] = (jnp.dot(h, l2w[...], preferred_element_type=jnp.float32)
                  + l2b[...])

    row = pl.BlockSpec((BR, D), lambda i: (i, 0))
    wspec = pl.BlockSpec((D, D), lambda i: (0, 0))
    vspec = pl.BlockSpec((1, D), lambda i: (0, 0))
    return pl.pallas_call(
        body,
        grid=(N // BR,),
        in_specs=[row, row, row, row,
                  wspec, vspec, wspec, vspec, vspec,
                  pl.BlockSpec((D, H2), lambda i: (0, 0)),
                  pl.BlockSpec((1, H2), lambda i: (0, 0)),
                  pl.BlockSpec((H2, OUT), lambda i: (0, 0)),
                  pl.BlockSpec((1, OUT), lambda i: (0, 0))],
        out_specs=pl.BlockSpec((BR, OUT), lambda i: (i, 0)),
        out_shape=jax.ShapeDtypeStruct((N, OUT), jnp.float32),
    )(S1a, S1b, C_b, x_bert1,
      W1_db_l, b1_db_l.reshape(1, D), W1_db_r,
      bn1_g.reshape(1, D), bn1_b.reshape(1, D),
      lin1_W, lin1_b.reshape(1, H2), lin2_W, lin2_b.reshape(1, OUT))


def kernel(x_bert, x_doc, ei_bd, ei_db,
           W0_bd_l, b0_bd_l, W0_bd_r, W0_db_l, b0_db_l, W0_db_r,
           W1_bd_l, b1_bd_l, W1_bd_r, W1_db_l, b1_db_l, W1_db_r,
           bn0_g, bn0_b, bn1_g, bn1_b,
           lin1_W, lin1_b, lin2_W, lin2_b):
    eint_bd = ei_bd.reshape(2, E // CHUNK, CHUNK).swapaxes(0, 1).reshape(-1)
    eint_db = ei_db.reshape(2, E // CHUNK, CHUNK).swapaxes(0, 1).reshape(-1)
    zf = jnp.zeros((NP_, D), jnp.float32)

    S_d, C_d, S_b, C_b = _sc_layer0(x_bert, x_doc, eint_bd, eint_db, zf)

    x_doc1, x_bert1 = _tc_layer0(
        S_d, C_d, S_b, C_b, x_bert, x_doc,
        W0_bd_l, b0_bd_l, W0_bd_r, W0_db_l, b0_db_l, W0_db_r, bn0_g, bn0_b)

    S1a, S1b = _sc_layer1(x_doc1, eint_db, zf)

    return _tc_head(S1a, S1b, C_b, x_bert1,
                    W1_db_l, b1_db_l, W1_db_r, bn1_g, bn1_b,
                    lin1_W, lin1_b, lin2_W, lin2_b)


# speedup vs baseline: 1.1837x; 1.1837x over previous
import functools

import jax
import jax.numpy as jnp
from jax import lax
from jax.experimental import pallas as pl
from jax.experimental.pallas import tpu as pltpu
from jax.experimental.pallas import tpu_sc as plsc

N = 10000
D = 128
E = 320000
NC, NS = 2, 16
CHUNK = 40
CHUNK1 = 40
NP_ = 10240
RPT = NP_ // NS
EPS = 1e-5
BR = 1000


def _fill_ones(ref, chunk):
    v = jnp.ones((16,), jnp.float32)
    for i in range(chunk):
        for j in range(D // 16):
            ref[i, pl.ds(j * 16, 16)] = v


NBUF = 5


def _feature_pass(table, src, dst, acc, bufs, eoff, chunk, n):
    idx_s, idx_d, rows, sis, sid, sg, ssc = bufs
    G = n // NBUF

    def issue_idx(i, b):
        pltpu.async_copy(src.at[pl.ds(eoff(i), chunk)], idx_s[b], sis[b])
        pltpu.async_copy(dst.at[pl.ds(eoff(i), chunk)], idx_d[b], sid[b])

    def wait_sc(b):
        pltpu.make_async_copy(rows[b], acc.at[idx_d[b]], ssc[b]).wait()

    def wait_g(b):
        pltpu.make_async_copy(table.at[idx_s[b]], rows[b], sg[b]).wait()

    def launch_g(i, b):
        pltpu.make_async_copy(src.at[pl.ds(0, chunk)], idx_s[b],
                              sis[b]).wait()
        pltpu.async_copy(table.at[idx_s[b]], rows[b], sg[b])

    issue_idx(0, 0)
    issue_idx(1, 1)
    launch_g(0, 0)

    def group(g, carry):
        for b in range(NBUF):
            b1, b2 = (b + 1) % NBUF, (b + 2) % NBUF
            i = NBUF * g + b
            if b >= NBUF - 2:
                wait_sc(b2)
            else:
                @pl.when(g > 0)
                def _():
                    wait_sc(b2)
            if b < NBUF - 2:
                issue_idx(i + 2, b2)
            else:
                @pl.when(g < G - 1)
                def _():
                    issue_idx(i + 2, b2)
            if b < NBUF - 1:
                launch_g(i + 1, b1)
            else:
                @pl.when(g < G - 1)
                def _():
                    launch_g(i + 1, b1)
            wait_g(b)
            pltpu.make_async_copy(dst.at[pl.ds(0, chunk)], idx_d[b],
                                  sid[b]).wait()
            pltpu.async_copy(rows[b], acc.at[idx_d[b]], ssc[b], add=True)
        return carry

    lax.fori_loop(0, G, group, 0)
    for j in range(NBUF - 2):
        wait_sc((2 + j) % NBUF)


def _count_pass(dst, acc, ones, idx_d, sid, ssc, eoff, chunk, n):
    G = n // NBUF

    def wait_sc(b):
        pltpu.make_async_copy(ones, acc.at[idx_d[b]], ssc[b]).wait()

    pltpu.async_copy(dst.at[pl.ds(eoff(0), chunk)], idx_d[0], sid[0])

    def group(g, carry):
        for b in range(NBUF):
            b1 = (b + 1) % NBUF
            i = NBUF * g + b
            if b == NBUF - 1:
                wait_sc(b1)
            else:
                @pl.when(g > 0)
                def _():
                    wait_sc(b1)
            if b < NBUF - 1:
                pltpu.async_copy(dst.at[pl.ds(eoff(i + 1), chunk)],
                                 idx_d[b1], sid[b1])
            else:
                @pl.when(g < G - 1)
                def _():
                    pltpu.async_copy(dst.at[pl.ds(eoff(i + 1), chunk)],
                                     idx_d[b1], sid[b1])
            pltpu.make_async_copy(dst.at[pl.ds(0, chunk)], idx_d[b],
                                  sid[b]).wait()
            pltpu.async_copy(ones, acc.at[idx_d[b]], ssc[b], add=True)
        return carry

    lax.fori_loop(0, G, group, 0)
    for j in range(NBUF - 1):
        wait_sc((1 + j) % NBUF)


def _sc_scratch(chunk):
    vm = []
    for _ in range(2 * NBUF):
        vm.append(pltpu.VMEM((chunk,), jnp.int32))
    for _ in range(NBUF):
        vm.append(pltpu.VMEM((chunk, D), jnp.float32))
    for _ in range(4 * NBUF):
        vm.append(pltpu.SemaphoreType.DMA)
    vm.append(pltpu.VMEM_SHARED((NP_, D), jnp.float32))
    return vm


def _sc_layer0(x_bert, x_doc, src_bd, dst_bd, src_db, dst_db, zf):
    mesh = plsc.VectorSubcoreMesh(core_axis_name="c", subcore_axis_name="s")
    ept = E // NS
    n = ept // CHUNK

    @functools.partial(
        pl.kernel,
        out_type=[
            jax.ShapeDtypeStruct((NP_, D), jnp.float32),
            jax.ShapeDtypeStruct((NP_, D), jnp.float32),
            jax.ShapeDtypeStruct((NP_, D), jnp.float32),
            jax.ShapeDtypeStruct((NP_, D), jnp.float32),
        ],
        mesh=mesh,
        scratch_types=_sc_scratch(CHUNK),
    )
    def k(xb, xd, sbd, dbd, sdb, ddb, zfeat, o_sd, o_cd, o_sb, o_cb, *scr):
        idx_s = scr[0:NBUF]
        idx_d = scr[NBUF:2 * NBUF]
        rows = scr[2 * NBUF:3 * NBUF]
        sis = scr[3 * NBUF:4 * NBUF]
        sid = scr[4 * NBUF:5 * NBUF]
        sg = scr[5 * NBUF:6 * NBUF]
        ssc = scr[6 * NBUF:7 * NBUF]
        acc = scr[7 * NBUF]
        bufs = (idx_s, idx_d, rows, sis, sid, sg, ssc)
        cid = lax.axis_index("c")
        sid_ = lax.axis_index("s")
        r0 = sid_ * RPT
        pltpu.sync_copy(zfeat.at[pl.ds(r0, RPT)], acc.at[pl.ds(r0, RPT)])
        plsc.subcore_barrier()

        def export(dst_hbm):
            pltpu.sync_copy(acc.at[pl.ds(r0, RPT)],
                            dst_hbm.at[pl.ds(r0, RPT)])

        def eoff(i):
            return sid_ * ept + i * CHUNK

        for c_sel, (table, src, dst, o_s, o_c) in enumerate(
                ((xb, sbd, dbd, o_sd, o_cd), (xd, sdb, ddb, o_sb, o_cb))):
            @pl.when(cid == c_sel)
            def _():
                _feature_pass(table, src, dst, acc, bufs, eoff, CHUNK, n)
                plsc.subcore_barrier()
                export(o_s)
                plsc.subcore_barrier()
                pltpu.sync_copy(zfeat.at[pl.ds(r0, RPT)],
                                acc.at[pl.ds(r0, RPT)])
                _fill_ones(rows[0], CHUNK)
                plsc.subcore_barrier()
                _count_pass(dst, acc, rows[0], idx_d, sid, ssc,
                            eoff, CHUNK, n)
                plsc.subcore_barrier()
                export(o_c)

    return k(x_bert, x_doc, src_bd, dst_bd, src_db, dst_db, zf)


def _sc_layer1(x_doc1, src_db, dst_db, zf):
    mesh = plsc.VectorSubcoreMesh(core_axis_name="c", subcore_axis_name="s")
    ept = E // (NC * NS)
    n = ept // CHUNK1

    @functools.partial(
        pl.kernel,
        out_type=[jax.ShapeDtypeStruct((NP_, D), jnp.float32),
                  jax.ShapeDtypeStruct((NP_, D), jnp.float32)],
        mesh=mesh,
        scratch_types=_sc_scratch(CHUNK1),
    )
    def k(xd1, sdb, ddb, zfeat, o_s0, o_s1, *scr):
        idx_s = scr[0:NBUF]
        idx_d = scr[NBUF:2 * NBUF]
        rows = scr[2 * NBUF:3 * NBUF]
        sis = scr[3 * NBUF:4 * NBUF]
        sid = scr[4 * NBUF:5 * NBUF]
        sg = scr[5 * NBUF:6 * NBUF]
        ssc = scr[6 * NBUF:7 * NBUF]
        acc = scr[7 * NBUF]
        bufs = (idx_s, idx_d, rows, sis, sid, sg, ssc)
        cid = lax.axis_index("c")
        sid_ = lax.axis_index("s")
        r0 = sid_ * RPT
        pltpu.sync_copy(zfeat.at[pl.ds(r0, RPT)], acc.at[pl.ds(r0, RPT)])
        plsc.subcore_barrier()

        wid = cid * NS + sid_

        def eoff(i):
            return wid * ept + i * CHUNK1

        _feature_pass(xd1, sdb, ddb, acc, bufs, eoff, CHUNK1, n)
        plsc.subcore_barrier()

        for c_sel, o_s in enumerate((o_s0, o_s1)):
            @pl.when(cid == c_sel)
            def _():
                pltpu.sync_copy(acc.at[pl.ds(r0, RPT)],
                                o_s.at[pl.ds(r0, RPT)])

    return k(x_doc1, src_db, dst_db, zf)


def _leaky(x):
    return jnp.where(x >= 0, x, 0.1 * x)


def _tc_layer0(S_d, C_d, S_b, C_b, x_bert, x_doc,
               W0_bd_l, b0_bd_l, W0_bd_r, W0_db_l, b0_db_l, W0_db_r,
               bn0_g, bn0_b):
    def body(sd, cd, sb, cb, xb, xd, wbl, bbl, wbr, wdl, bdl, wdr, g, b,
             o_xd1, o_xb1):
        scale = g[...] / jnp.sqrt(jnp.float32(1.0 + EPS))
        shift = b[...]
        cdv = jnp.maximum(cd[...][:, 0:1], 1.0)
        nd = (jnp.dot(sd[...], wbl[...], preferred_element_type=jnp.float32)
              / cdv + bbl[...]
              + jnp.dot(xd[...], wbr[...], preferred_element_type=jnp.float32))
        o_xd1[...] = _leaky(nd) * scale + shift
        cbv = jnp.maximum(cb[...][:, 0:1], 1.0)
        nb = (jnp.dot(sb[...], wdl[...], preferred_element_type=jnp.float32)
              / cbv + bdl[...]
              + jnp.dot(xb[...], wdr[...], preferred_element_type=jnp.float32))
        o_xb1[...] = _leaky(nb) * scale + shift

    row = pl.BlockSpec((BR, D), lambda i: (i, 0))
    wspec = pl.BlockSpec((D, D), lambda i: (0, 0))
    vspec = pl.BlockSpec((1, D), lambda i: (0, 0))
    return pl.pallas_call(
        body,
        grid=(N // BR,),
        in_specs=[row, row, row, row, row, row,
                  wspec, vspec, wspec, wspec, vspec, wspec, vspec, vspec],
        out_specs=[row, row],
        out_shape=[jax.ShapeDtypeStruct((N, D), jnp.float32),
                   jax.ShapeDtypeStruct((N, D), jnp.float32)],
    )(S_d, C_d, S_b, C_b, x_bert, x_doc,
      W0_bd_l, b0_bd_l.reshape(1, D), W0_bd_r,
      W0_db_l, b0_db_l.reshape(1, D), W0_db_r,
      bn0_g.reshape(1, D), bn0_b.reshape(1, D))


def _tc_head(S1a, S1b, C_b, x_bert1,
             W1_db_l, b1_db_l, W1_db_r, bn1_g, bn1_b,
             lin1_W, lin1_b, lin2_W, lin2_b):
    H2 = D // 2
    OUT = 16

    def body(s0, s1, cb, xb1, wdl, bdl, wdr, g, b, l1w, l1b, l2w, l2b, o):
        scale = g[...] / jnp.sqrt(jnp.float32(1.0 + EPS))
        cbv = jnp.maximum(cb[...][:, 0:1], 1.0)
        s = s0[...] + s1[...]
        nb = (jnp.dot(s, wdl[...], preferred_element_type=jnp.float32)
              / cbv + bdl[...]
              + jnp.dot(xb1[...], wdr[...], preferred_element_type=jnp.float32))
        xb2 = _leaky(nb) * scale + b[...]
        h = _leaky(jnp.dot(xb2, l1w[...], preferred_element_type=jnp.float32)
                   + l1b[...])
        o[...] = (jnp.dot(h, l2w[...], preferred_element_type=jnp.float32)
                  + l2b[...])

    row = pl.BlockSpec((BR, D), lambda i: (i, 0))
    wspec = pl.BlockSpec((D, D), lambda i: (0, 0))
    vspec = pl.BlockSpec((1, D), lambda i: (0, 0))
    return pl.pallas_call(
        body,
        grid=(N // BR,),
        in_specs=[row, row, row, row,
                  wspec, vspec, wspec, vspec, vspec,
                  pl.BlockSpec((D, H2), lambda i: (0, 0)),
                  pl.BlockSpec((1, H2), lambda i: (0, 0)),
                  pl.BlockSpec((H2, OUT), lambda i: (0, 0)),
                  pl.BlockSpec((1, OUT), lambda i: (0, 0))],
        out_specs=pl.BlockSpec((BR, OUT), lambda i: (i, 0)),
        out_shape=jax.ShapeDtypeStruct((N, OUT), jnp.float32),
    )(S1a, S1b, C_b, x_bert1,
      W1_db_l, b1_db_l.reshape(1, D), W1_db_r,
      bn1_g.reshape(1, D), bn1_b.reshape(1, D),
      lin1_W, lin1_b.reshape(1, H2), lin2_W, lin2_b.reshape(1, OUT))


def kernel(x_bert, x_doc, ei_bd, ei_db,
           W0_bd_l, b0_bd_l, W0_bd_r, W0_db_l, b0_db_l, W0_db_r,
           W1_bd_l, b1_bd_l, W1_bd_r, W1_db_l, b1_db_l, W1_db_r,
           bn0_g, bn0_b, bn1_g, bn1_b,
           lin1_W, lin1_b, lin2_W, lin2_b):
    src_bd, dst_bd = ei_bd[0], ei_bd[1]
    src_db, dst_db = ei_db[0], ei_db[1]
    zf = jnp.zeros((NP_, D), jnp.float32)

    S_d, C_d, S_b, C_b = _sc_layer0(
        x_bert, x_doc, src_bd, dst_bd, src_db, dst_db, zf)

    x_doc1, x_bert1 = _tc_layer0(
        S_d, C_d, S_b, C_b, x_bert, x_doc,
        W0_bd_l, b0_bd_l, W0_bd_r, W0_db_l, b0_db_l, W0_db_r, bn0_g, bn0_b)

    S1a, S1b = _sc_layer1(x_doc1, src_db, dst_db, zf)

    return _tc_head(S1a, S1b, C_b, x_bert1,
                    W1_db_l, b1_db_l, W1_db_r, bn1_g, bn1_b,
                    lin1_W, lin1_b, lin2_W, lin2_b)


# speedup vs baseline: 1.1906x; 1.0058x over previous
import functools

import jax
import jax.numpy as jnp
from jax import lax
from jax.experimental import pallas as pl
from jax.experimental.pallas import tpu as pltpu
from jax.experimental.pallas import tpu_sc as plsc

N = 10000
D = 128
E = 320000
NC, NS = 2, 16
CHUNK = 40
CHUNK1 = 40
NP_ = 10240
RPT = NP_ // NS
EPS = 1e-5
BR = 1000


def _fill_ones(ref, chunk):
    v = jnp.ones((16,), jnp.float32)
    for i in range(chunk):
        for j in range(D // 16):
            ref[i, pl.ds(j * 16, 16)] = v


NBUF = 5


def _feature_pass(table, src, dst, acc, bufs, eoff, chunk, n):
    idx_s, idx_d, rows, sis, sid, sg, ssc = bufs
    G = n // NBUF

    def issue_idx(i, b):
        pltpu.async_copy(src.at[pl.ds(eoff(i), chunk)], idx_s[b], sis[b])
        pltpu.async_copy(dst.at[pl.ds(eoff(i), chunk)], idx_d[b], sid[b])

    def wait_sc(b):
        pltpu.make_async_copy(rows[b], acc.at[idx_d[b]], ssc[b]).wait()

    def wait_g(b):
        pltpu.make_async_copy(table.at[idx_s[b]], rows[b], sg[b]).wait()

    def launch_g(i, b):
        pltpu.make_async_copy(src.at[pl.ds(0, chunk)], idx_s[b],
                              sis[b]).wait()
        pltpu.async_copy(table.at[idx_s[b]], rows[b], sg[b])

    issue_idx(0, 0)
    issue_idx(1, 1)
    launch_g(0, 0)

    def group(g, carry):
        for b in range(NBUF):
            b1, b2 = (b + 1) % NBUF, (b + 2) % NBUF
            i = NBUF * g + b
            if b >= NBUF - 2:
                wait_sc(b2)
            else:
                @pl.when(g > 0)
                def _():
                    wait_sc(b2)
            if b < NBUF - 2:
                issue_idx(i + 2, b2)
            else:
                @pl.when(g < G - 1)
                def _():
                    issue_idx(i + 2, b2)
            if b < NBUF - 1:
                launch_g(i + 1, b1)
            else:
                @pl.when(g < G - 1)
                def _():
                    launch_g(i + 1, b1)
            wait_g(b)
            pltpu.make_async_copy(dst.at[pl.ds(0, chunk)], idx_d[b],
                                  sid[b]).wait()
            pltpu.async_copy(rows[b], acc.at[idx_d[b]], ssc[b], add=True)
        return carry

    lax.fori_loop(0, G, group, 0)
    for j in range(NBUF - 2):
        wait_sc((2 + j) % NBUF)


def _count_pass(dst, acc, ones, idx_d, sid, ssc, eoff, chunk, n):
    G = n // NBUF

    def wait_sc(b):
        pltpu.make_async_copy(ones, acc.at[idx_d[b]], ssc[b]).wait()

    pltpu.async_copy(dst.at[pl.ds(eoff(0), chunk)], idx_d[0], sid[0])

    def group(g, carry):
        for b in range(NBUF):
            b1 = (b + 1) % NBUF
            i = NBUF * g + b
            if b == NBUF - 1:
                wait_sc(b1)
            else:
                @pl.when(g > 0)
                def _():
                    wait_sc(b1)
            if b < NBUF - 1:
                pltpu.async_copy(dst.at[pl.ds(eoff(i + 1), chunk)],
                                 idx_d[b1], sid[b1])
            else:
                @pl.when(g < G - 1)
                def _():
                    pltpu.async_copy(dst.at[pl.ds(eoff(i + 1), chunk)],
                                     idx_d[b1], sid[b1])
            pltpu.make_async_copy(dst.at[pl.ds(0, chunk)], idx_d[b],
                                  sid[b]).wait()
            pltpu.async_copy(ones, acc.at[idx_d[b]], ssc[b], add=True)
        return carry

    lax.fori_loop(0, G, group, 0)
    for j in range(NBUF - 1):
        wait_sc((1 + j) % NBUF)


def _sc_scratch(chunk):
    vm = []
    for _ in range(2 * NBUF):
        vm.append(pltpu.VMEM((chunk,), jnp.int32))
    for _ in range(NBUF):
        vm.append(pltpu.VMEM((chunk, D), jnp.float32))
    for _ in range(4 * NBUF):
        vm.append(pltpu.SemaphoreType.DMA)
    vm.append(pltpu.VMEM_SHARED((NP_, D), jnp.float32))
    return vm


def _sc_layer0(x_bert, x_doc, src_bd, dst_bd, src_db, dst_db, zf):
    mesh = plsc.VectorSubcoreMesh(core_axis_name="c", subcore_axis_name="s")
    ept = E // NS
    n = ept // CHUNK

    @functools.partial(
        pl.kernel,
        out_type=[
            jax.ShapeDtypeStruct((NP_, D), jnp.float32),
            jax.ShapeDtypeStruct((NP_, D), jnp.float32),
            jax.ShapeDtypeStruct((NP_, D), jnp.float32),
            jax.ShapeDtypeStruct((NP_, D), jnp.float32),
        ],
        mesh=mesh,
        scratch_types=_sc_scratch(CHUNK),
    )
    def k(xb, xd, sbd, dbd, sdb, ddb, zfeat, o_sd, o_cd, o_sb, o_cb, *scr):
        idx_s = scr[0:NBUF]
        idx_d = scr[NBUF:2 * NBUF]
        rows = scr[2 * NBUF:3 * NBUF]
        sis = scr[3 * NBUF:4 * NBUF]
        sid = scr[4 * NBUF:5 * NBUF]
        sg = scr[5 * NBUF:6 * NBUF]
        ssc = scr[6 * NBUF:7 * NBUF]
        acc = scr[7 * NBUF]
        bufs = (idx_s, idx_d, rows, sis, sid, sg, ssc)
        cid = lax.axis_index("c")
        sid_ = lax.axis_index("s")
        r0 = sid_ * RPT
        pltpu.sync_copy(zfeat.at[pl.ds(r0, RPT)], acc.at[pl.ds(r0, RPT)])
        plsc.subcore_barrier()

        def export(dst_hbm):
            pltpu.sync_copy(acc.at[pl.ds(r0, RPT)],
                            dst_hbm.at[pl.ds(r0, RPT)])

        def eoff(i):
            return sid_ * ept + i * CHUNK

        for c_sel, (table, src, dst, o_s, o_c) in enumerate(
                ((xb, sbd, dbd, o_sd, o_cd), (xd, sdb, ddb, o_sb, o_cb))):
            @pl.when(cid == c_sel)
            def _():
                _feature_pass(table, src, dst, acc, bufs, eoff, CHUNK, n)
                plsc.subcore_barrier()
                export(o_s)
                plsc.subcore_barrier()
                pltpu.sync_copy(zfeat.at[pl.ds(r0, RPT)],
                                acc.at[pl.ds(r0, RPT)])
                _fill_ones(rows[0], CHUNK)
                plsc.subcore_barrier()
                _count_pass(dst, acc, rows[0], idx_d, sid, ssc,
                            eoff, CHUNK, n)
                plsc.subcore_barrier()
                export(o_c)

    return k(x_bert, x_doc, src_bd, dst_bd, src_db, dst_db, zf)


def _sc_layer1(x_doc1, src_db, dst_db, zf):
    mesh = plsc.VectorSubcoreMesh(core_axis_name="c", subcore_axis_name="s")
    ept = E // (NC * NS)
    n = ept // CHUNK1

    @functools.partial(
        pl.kernel,
        out_type=[jax.ShapeDtypeStruct((NP_, D), jnp.float32),
                  jax.ShapeDtypeStruct((NP_, D), jnp.float32)],
        mesh=mesh,
        scratch_types=_sc_scratch(CHUNK1),
    )
    def k(xd1, sdb, ddb, zfeat, o_s0, o_s1, *scr):
        idx_s = scr[0:NBUF]
        idx_d = scr[NBUF:2 * NBUF]
        rows = scr[2 * NBUF:3 * NBUF]
        sis = scr[3 * NBUF:4 * NBUF]
        sid = scr[4 * NBUF:5 * NBUF]
        sg = scr[5 * NBUF:6 * NBUF]
        ssc = scr[6 * NBUF:7 * NBUF]
        acc = scr[7 * NBUF]
        bufs = (idx_s, idx_d, rows, sis, sid, sg, ssc)
        cid = lax.axis_index("c")
        sid_ = lax.axis_index("s")
        r0 = sid_ * RPT
        pltpu.sync_copy(zfeat.at[pl.ds(r0, RPT)], acc.at[pl.ds(r0, RPT)])
        plsc.subcore_barrier()

        wid = cid * NS + sid_

        def eoff(i):
            return wid * ept + i * CHUNK1

        _feature_pass(xd1, sdb, ddb, acc, bufs, eoff, CHUNK1, n)
        plsc.subcore_barrier()

        for c_sel, o_s in enumerate((o_s0, o_s1)):
            @pl.when(cid == c_sel)
            def _():
                pltpu.sync_copy(acc.at[pl.ds(r0, RPT)],
                                o_s.at[pl.ds(r0, RPT)])

    return k(x_doc1, src_db, dst_db, zf)


def _leaky(x):
    return jnp.where(x >= 0, x, 0.1 * x)


def _tc_half(S, C, x_dst, Wl, bl, Wr, g, b):
    def body(s, c, xd, wl, bl_, wr, g_, b_, o):
        scale = g_[...] / jnp.sqrt(jnp.float32(1.0 + EPS))
        cv = jnp.maximum(c[...][:, 0:1], 1.0)
        nd = (jnp.dot(s[...], wl[...], preferred_element_type=jnp.float32)
              / cv + bl_[...]
              + jnp.dot(xd[...], wr[...], preferred_element_type=jnp.float32))
        o[...] = jnp.where(nd >= 0, nd, 0.1 * nd) * scale + b_[...]

    row = pl.BlockSpec((BR, D), lambda i: (i, 0))
    wspec = pl.BlockSpec((D, D), lambda i: (0, 0))
    vspec = pl.BlockSpec((1, D), lambda i: (0, 0))
    return pl.pallas_call(
        body,
        grid=(N // BR,),
        in_specs=[row, row, row, wspec, vspec, wspec, vspec, vspec],
        out_specs=row,
        out_shape=jax.ShapeDtypeStruct((N, D), jnp.float32),
    )(S, C, x_dst, Wl, bl.reshape(1, D), Wr,
      g.reshape(1, D), b.reshape(1, D))


def _tc_head(S1a, S1b, C_b, x_bert1,
             W1_db_l, b1_db_l, W1_db_r, bn1_g, bn1_b,
             lin1_W, lin1_b, lin2_W, lin2_b):
    H2 = D // 2
    OUT = 16

    def body(s0, s1, cb, xb1, wdl, bdl, wdr, g, b, l1w, l1b, l2w, l2b, o):
        scale = g[...] / jnp.sqrt(jnp.float32(1.0 + EPS))
        cbv = jnp.maximum(cb[...][:, 0:1], 1.0)
        s = s0[...] + s1[...]
        nb = (jnp.dot(s, wdl[...], preferred_element_type=jnp.float32)
              / cbv + bdl[...]
              + jnp.dot(xb1[...], wdr[...], preferred_element_type=jnp.float32))
        xb2 = _leaky(nb) * scale + b[...]
        h = _leaky(jnp.dot(xb2, l1w[...], preferred_element_type=jnp.float32)
                   + l1b[...])
        o[...] = (jnp.dot(h, l2w[...], preferred_element_type=jnp.float32)
                  + l2b[...])

    row = pl.BlockSpec((BR, D), lambda i: (i, 0))
    wspec = pl.BlockSpec((D, D), lambda i: (0, 0))
    vspec = pl.BlockSpec((1, D), lambda i: (0, 0))
    return pl.pallas_call(
        body,
        grid=(N // BR,),
        in_specs=[row, row, row, row,
                  wspec, vspec, wspec, vspec, vspec,
                  pl.BlockSpec((D, H2), lambda i: (0, 0)),
                  pl.BlockSpec((1, H2), lambda i: (0, 0)),
                  pl.BlockSpec((H2, OUT), lambda i: (0, 0)),
                  pl.BlockSpec((1, OUT), lambda i: (0, 0))],
        out_specs=pl.BlockSpec((BR, OUT), lambda i: (i, 0)),
        out_shape=jax.ShapeDtypeStruct((N, OUT), jnp.float32),
    )(S1a, S1b, C_b, x_bert1,
      W1_db_l, b1_db_l.reshape(1, D), W1_db_r,
      bn1_g.reshape(1, D), bn1_b.reshape(1, D),
      lin1_W, lin1_b.reshape(1, H2), lin2_W, lin2_b.reshape(1, OUT))


def kernel(x_bert, x_doc, ei_bd, ei_db,
           W0_bd_l, b0_bd_l, W0_bd_r, W0_db_l, b0_db_l, W0_db_r,
           W1_bd_l, b1_bd_l, W1_bd_r, W1_db_l, b1_db_l, W1_db_r,
           bn0_g, bn0_b, bn1_g, bn1_b,
           lin1_W, lin1_b, lin2_W, lin2_b):
    src_bd, dst_bd = ei_bd[0], ei_bd[1]
    src_db, dst_db = ei_db[0], ei_db[1]
    zf = jnp.zeros((NP_, D), jnp.float32)

    S_d, C_d, S_b, C_b = _sc_layer0(
        x_bert, x_doc, src_bd, dst_bd, src_db, dst_db, zf)

    x_doc1 = _tc_half(S_d, C_d, x_doc, W0_bd_l, b0_bd_l, W0_bd_r,
                      bn0_g, bn0_b)

    S1a, S1b = _sc_layer1(x_doc1, src_db, dst_db, zf)

    x_bert1 = _tc_half(S_b, C_b, x_bert, W0_db_l, b0_db_l, W0_db_r,
                       bn0_g, bn0_b)

    return _tc_head(S1a, S1b, C_b, x_bert1,
                    W1_db_l, b1_db_l, W1_db_r, bn1_g, bn1_b,
                    lin1_W, lin1_b, lin2_W, lin2_b)


# speedup vs baseline: 1.2044x; 1.0116x over previous
import functools

import jax
import jax.numpy as jnp
from jax import lax
from jax.experimental import pallas as pl
from jax.experimental.pallas import tpu as pltpu
from jax.experimental.pallas import tpu_sc as plsc

N = 10000
D = 128
E = 320000
NC, NS = 2, 16
CHUNK = 40
CHUNK1 = 40
NP_ = 10240
RPT = NP_ // NS
EPS = 1e-5
BR = 2000


def _fill_ones(ref, chunk):
    v = jnp.ones((16,), jnp.float32)
    for i in range(chunk):
        for j in range(D // 16):
            ref[i, pl.ds(j * 16, 16)] = v


NBUF = 5


def _feature_pass(table, src, dst, acc, bufs, eoff, chunk, n):
    idx_s, idx_d, rows, sis, sid, sg, ssc = bufs
    G = n // NBUF

    def issue_idx(i, b):
        pltpu.async_copy(src.at[pl.ds(eoff(i), chunk)], idx_s[b], sis[b])
        pltpu.async_copy(dst.at[pl.ds(eoff(i), chunk)], idx_d[b], sid[b])

    def wait_sc(b):
        pltpu.make_async_copy(rows[b], acc.at[idx_d[b]], ssc[b]).wait()

    def wait_g(b):
        pltpu.make_async_copy(table.at[idx_s[b]], rows[b], sg[b]).wait()

    def launch_g(i, b):
        pltpu.make_async_copy(src.at[pl.ds(0, chunk)], idx_s[b],
                              sis[b]).wait()
        pltpu.async_copy(table.at[idx_s[b]], rows[b], sg[b])

    issue_idx(0, 0)
    issue_idx(1, 1)
    launch_g(0, 0)

    def group(g, carry):
        for b in range(NBUF):
            b1, b2 = (b + 1) % NBUF, (b + 2) % NBUF
            i = NBUF * g + b
            if b >= NBUF - 2:
                wait_sc(b2)
            else:
                @pl.when(g > 0)
                def _():
                    wait_sc(b2)
            if b < NBUF - 2:
                issue_idx(i + 2, b2)
            else:
                @pl.when(g < G - 1)
                def _():
                    issue_idx(i + 2, b2)
            if b < NBUF - 1:
                launch_g(i + 1, b1)
            else:
                @pl.when(g < G - 1)
                def _():
                    launch_g(i + 1, b1)
            wait_g(b)
            pltpu.make_async_copy(dst.at[pl.ds(0, chunk)], idx_d[b],
                                  sid[b]).wait()
            pltpu.async_copy(rows[b], acc.at[idx_d[b]], ssc[b], add=True)
        return carry

    lax.fori_loop(0, G, group, 0)
    for j in range(NBUF - 2):
        wait_sc((2 + j) % NBUF)


def _count_pass(dst, acc, ones, idx_d, sid, ssc, eoff, chunk, n):
    G = n // NBUF

    def wait_sc(b):
        pltpu.make_async_copy(ones, acc.at[idx_d[b]], ssc[b]).wait()

    pltpu.async_copy(dst.at[pl.ds(eoff(0), chunk)], idx_d[0], sid[0])

    def group(g, carry):
        for b in range(NBUF):
            b1 = (b + 1) % NBUF
            i = NBUF * g + b
            if b == NBUF - 1:
                wait_sc(b1)
            else:
                @pl.when(g > 0)
                def _():
                    wait_sc(b1)
            if b < NBUF - 1:
                pltpu.async_copy(dst.at[pl.ds(eoff(i + 1), chunk)],
                                 idx_d[b1], sid[b1])
            else:
                @pl.when(g < G - 1)
                def _():
                    pltpu.async_copy(dst.at[pl.ds(eoff(i + 1), chunk)],
                                     idx_d[b1], sid[b1])
            pltpu.make_async_copy(dst.at[pl.ds(0, chunk)], idx_d[b],
                                  sid[b]).wait()
            pltpu.async_copy(ones, acc.at[idx_d[b]], ssc[b], add=True)
        return carry

    lax.fori_loop(0, G, group, 0)
    for j in range(NBUF - 1):
        wait_sc((1 + j) % NBUF)


def _sc_scratch(chunk):
    vm = []
    for _ in range(2 * NBUF):
        vm.append(pltpu.VMEM((chunk,), jnp.int32))
    for _ in range(NBUF):
        vm.append(pltpu.VMEM((chunk, D), jnp.float32))
    for _ in range(4 * NBUF):
        vm.append(pltpu.SemaphoreType.DMA)
    vm.append(pltpu.VMEM_SHARED((NP_, D), jnp.float32))
    return vm


def _sc_layer0(x_bert, x_doc, src_bd, dst_bd, src_db, dst_db, zf):
    mesh = plsc.VectorSubcoreMesh(core_axis_name="c", subcore_axis_name="s")
    ept = E // NS
    n = ept // CHUNK

    @functools.partial(
        pl.kernel,
        out_type=[
            jax.ShapeDtypeStruct((NP_, D), jnp.float32),
            jax.ShapeDtypeStruct((NP_, D), jnp.float32),
            jax.ShapeDtypeStruct((NP_, D), jnp.float32),
            jax.ShapeDtypeStruct((NP_, D), jnp.float32),
        ],
        mesh=mesh,
        scratch_types=_sc_scratch(CHUNK),
    )
    def k(xb, xd, sbd, dbd, sdb, ddb, zfeat, o_sd, o_cd, o_sb, o_cb, *scr):
        idx_s = scr[0:NBUF]
        idx_d = scr[NBUF:2 * NBUF]
        rows = scr[2 * NBUF:3 * NBUF]
        sis = scr[3 * NBUF:4 * NBUF]
        sid = scr[4 * NBUF:5 * NBUF]
        sg = scr[5 * NBUF:6 * NBUF]
        ssc = scr[6 * NBUF:7 * NBUF]
        acc = scr[7 * NBUF]
        bufs = (idx_s, idx_d, rows, sis, sid, sg, ssc)
        cid = lax.axis_index("c")
        sid_ = lax.axis_index("s")
        r0 = sid_ * RPT
        pltpu.sync_copy(zfeat.at[pl.ds(r0, RPT)], acc.at[pl.ds(r0, RPT)])
        plsc.subcore_barrier()

        def export(dst_hbm):
            pltpu.sync_copy(acc.at[pl.ds(r0, RPT)],
                            dst_hbm.at[pl.ds(r0, RPT)])

        def eoff(i):
            return sid_ * ept + i * CHUNK

        for c_sel, (table, src, dst, o_s, o_c) in enumerate(
                ((xb, sbd, dbd, o_sd, o_cd), (xd, sdb, ddb, o_sb, o_cb))):
            @pl.when(cid == c_sel)
            def _():
                _feature_pass(table, src, dst, acc, bufs, eoff, CHUNK, n)
                plsc.subcore_barrier()
                export(o_s)
                plsc.subcore_barrier()
                pltpu.sync_copy(zfeat.at[pl.ds(r0, RPT)],
                                acc.at[pl.ds(r0, RPT)])
                _fill_ones(rows[0], CHUNK)
                plsc.subcore_barrier()
                _count_pass(dst, acc, rows[0], idx_d, sid, ssc,
                            eoff, CHUNK, n)
                plsc.subcore_barrier()
                export(o_c)

    return k(x_bert, x_doc, src_bd, dst_bd, src_db, dst_db, zf)


def _sc_layer1(x_doc1, src_db, dst_db, zf):
    mesh = plsc.VectorSubcoreMesh(core_axis_name="c", subcore_axis_name="s")
    ept = E // (NC * NS)
    n = ept // CHUNK1

    @functools.partial(
        pl.kernel,
        out_type=[jax.ShapeDtypeStruct((NP_, D), jnp.float32),
                  jax.ShapeDtypeStruct((NP_, D), jnp.float32)],
        mesh=mesh,
        scratch_types=_sc_scratch(CHUNK1),
    )
    def k(xd1, sdb, ddb, zfeat, o_s0, o_s1, *scr):
        idx_s = scr[0:NBUF]
        idx_d = scr[NBUF:2 * NBUF]
        rows = scr[2 * NBUF:3 * NBUF]
        sis = scr[3 * NBUF:4 * NBUF]
        sid = scr[4 * NBUF:5 * NBUF]
        sg = scr[5 * NBUF:6 * NBUF]
        ssc = scr[6 * NBUF:7 * NBUF]
        acc = scr[7 * NBUF]
        bufs = (idx_s, idx_d, rows, sis, sid, sg, ssc)
        cid = lax.axis_index("c")
        sid_ = lax.axis_index("s")
        r0 = sid_ * RPT
        pltpu.sync_copy(zfeat.at[pl.ds(r0, RPT)], acc.at[pl.ds(r0, RPT)])
        plsc.subcore_barrier()

        wid = cid * NS + sid_

        def eoff(i):
            return wid * ept + i * CHUNK1

        _feature_pass(xd1, sdb, ddb, acc, bufs, eoff, CHUNK1, n)
        plsc.subcore_barrier()

        for c_sel, o_s in enumerate((o_s0, o_s1)):
            @pl.when(cid == c_sel)
            def _():
                pltpu.sync_copy(acc.at[pl.ds(r0, RPT)],
                                o_s.at[pl.ds(r0, RPT)])

    return k(x_doc1, src_db, dst_db, zf)


def _leaky(x):
    return jnp.where(x >= 0, x, 0.1 * x)


def _tc_half(S, C, x_dst, Wl, bl, Wr, g, b):
    def body(s, c, xd, wl, bl_, wr, g_, b_, o):
        scale = g_[...] / jnp.sqrt(jnp.float32(1.0 + EPS))
        cv = jnp.maximum(c[...][:, 0:1], 1.0)
        nd = (jnp.dot(s[...], wl[...], preferred_element_type=jnp.float32)
              / cv + bl_[...]
              + jnp.dot(xd[...], wr[...], preferred_element_type=jnp.float32))
        o[...] = jnp.where(nd >= 0, nd, 0.1 * nd) * scale + b_[...]

    row = pl.BlockSpec((BR, D), lambda i: (i, 0))
    wspec = pl.BlockSpec((D, D), lambda i: (0, 0))
    vspec = pl.BlockSpec((1, D), lambda i: (0, 0))
    return pl.pallas_call(
        body,
        grid=(N // BR,),
        in_specs=[row, row, row, wspec, vspec, wspec, vspec, vspec],
        out_specs=row,
        out_shape=jax.ShapeDtypeStruct((N, D), jnp.float32),
    )(S, C, x_dst, Wl, bl.reshape(1, D), Wr,
      g.reshape(1, D), b.reshape(1, D))


def _tc_head(S1a, S1b, C_b, x_bert1,
             W1_db_l, b1_db_l, W1_db_r, bn1_g, bn1_b,
             lin1_W, lin1_b, lin2_W, lin2_b):
    H2 = D // 2
    OUT = 16

    def body(s0, s1, cb, xb1, wdl, bdl, wdr, g, b, l1w, l1b, l2w, l2b, o):
        scale = g[...] / jnp.sqrt(jnp.float32(1.0 + EPS))
        cbv = jnp.maximum(cb[...][:, 0:1], 1.0)
        s = s0[...] + s1[...]
        nb = (jnp.dot(s, wdl[...], preferred_element_type=jnp.float32)
              / cbv + bdl[...]
              + jnp.dot(xb1[...], wdr[...], preferred_element_type=jnp.float32))
        xb2 = _leaky(nb) * scale + b[...]
        h = _leaky(jnp.dot(xb2, l1w[...], preferred_element_type=jnp.float32)
                   + l1b[...])
        o[...] = (jnp.dot(h, l2w[...], preferred_element_type=jnp.float32)
                  + l2b[...])

    row = pl.BlockSpec((BR, D), lambda i: (i, 0))
    wspec = pl.BlockSpec((D, D), lambda i: (0, 0))
    vspec = pl.BlockSpec((1, D), lambda i: (0, 0))
    return pl.pallas_call(
        body,
        grid=(N // BR,),
        in_specs=[row, row, row, row,
                  wspec, vspec, wspec, vspec, vspec,
                  pl.BlockSpec((D, H2), lambda i: (0, 0)),
                  pl.BlockSpec((1, H2), lambda i: (0, 0)),
                  pl.BlockSpec((H2, OUT), lambda i: (0, 0)),
                  pl.BlockSpec((1, OUT), lambda i: (0, 0))],
        out_specs=pl.BlockSpec((BR, OUT), lambda i: (i, 0)),
        out_shape=jax.ShapeDtypeStruct((N, OUT), jnp.float32),
    )(S1a, S1b, C_b, x_bert1,
      W1_db_l, b1_db_l.reshape(1, D), W1_db_r,
      bn1_g.reshape(1, D), bn1_b.reshape(1, D),
      lin1_W, lin1_b.reshape(1, H2), lin2_W, lin2_b.reshape(1, OUT))


def kernel(x_bert, x_doc, ei_bd, ei_db,
           W0_bd_l, b0_bd_l, W0_bd_r, W0_db_l, b0_db_l, W0_db_r,
           W1_bd_l, b1_bd_l, W1_bd_r, W1_db_l, b1_db_l, W1_db_r,
           bn0_g, bn0_b, bn1_g, bn1_b,
           lin1_W, lin1_b, lin2_W, lin2_b):
    src_bd, dst_bd = ei_bd[0], ei_bd[1]
    src_db, dst_db = ei_db[0], ei_db[1]
    zf = jnp.zeros((NP_, D), jnp.float32)

    S_d, C_d, S_b, C_b = _sc_layer0(
        x_bert, x_doc, src_bd, dst_bd, src_db, dst_db, zf)

    x_doc1 = _tc_half(S_d, C_d, x_doc, W0_bd_l, b0_bd_l, W0_bd_r,
                      bn0_g, bn0_b)

    S1a, S1b = _sc_layer1(x_doc1, src_db, dst_db, zf)

    x_bert1 = _tc_half(S_b, C_b, x_bert, W0_db_l, b0_db_l, W0_db_r,
                       bn0_g, bn0_b)

    return _tc_head(S1a, S1b, C_b, x_bert1,
                    W1_db_l, b1_db_l, W1_db_r, bn1_g, bn1_b,
                    lin1_W, lin1_b, lin2_W, lin2_b)
